# Initial kernel scaffold; baseline (speedup 1.0000x reference)
import jax
import jax.numpy as jnp
from jax.experimental import pallas as pl


def kernel(query, key, offset, rel_bias_table):
    raise NotImplementedError("write your pallas kernel here")


# speedup vs baseline: 72.2919x; 72.2919x over previous
import functools
import math

import jax
import jax.numpy as jnp
from jax.experimental import pallas as pl
from jax.experimental.pallas import tpu as pltpu

_TI = 128


def _bias_tile_kernel(off_ref, tab_ref, out_ref, *, num_buckets, max_distance):
    ti = out_ref.shape[2]
    s_len = out_ref.shape[3]
    strip_len = ti + s_len - 1
    strip_pad = ((strip_len + 127) // 128) * 128
    t = pl.program_id(1)
    i0 = t * ti

    s_idx = jax.lax.broadcasted_iota(jnp.int32, (1, strip_pad), 1)
    d = jnp.maximum(i0 + off_ref[0] + (ti - 1) - s_idx, 0)

    max_exact = num_buckets // 2
    is_small = d < max_exact
    d_f = d.astype(jnp.float32)
    large = max_exact + (
        jnp.log(d_f / max_exact)
        / math.log(max_distance / max_exact)
        * (num_buckets - max_exact)
    ).astype(jnp.int32)
    large = jnp.minimum(large, num_buckets - 1)
    bucket = jnp.where(is_small, d, large)

    h = pl.program_id(0)
    val = jnp.zeros((1, strip_pad), jnp.float32)
    for b in range(num_buckets):
        val = jnp.where(bucket == b, tab_ref[b, h], val)

    rows = []
    for ii in range(ti):
        st = ti - 1 - ii
        rows.append(jax.lax.slice(val, (0, st), (1, st + s_len)))
        if len(rows) == 8:
            out_ref[0, 0, ii - 7 : ii + 1, :] = jnp.concatenate(rows, axis=0)
            rows = []


def kernel(query, key, offset, rel_bias_table):
    t_len = query.shape[1]
    s_len = key.shape[1]
    num_buckets, num_heads = rel_bias_table.shape
    off = jnp.asarray(offset, jnp.int32).reshape(1)

    body = functools.partial(
        _bias_tile_kernel, num_buckets=num_buckets, max_distance=128
    )
    return pl.pallas_call(
        body,
        grid=(num_heads, t_len // _TI),
        in_specs=[
            pl.BlockSpec(memory_space=pltpu.SMEM),
            pl.BlockSpec(memory_space=pltpu.SMEM),
        ],
        out_specs=pl.BlockSpec((1, 1, _TI, s_len), lambda h, t: (0, h, t, 0)),
        out_shape=jax.ShapeDtypeStruct((1, num_heads, t_len, s_len), jnp.float32),
    )(off, rel_bias_table)


# speedup vs baseline: 77.6360x; 1.0739x over previous
import functools
import math

import jax
import jax.numpy as jnp
from jax.experimental import pallas as pl
from jax.experimental.pallas import tpu as pltpu

_TI = 128


def _diag_vals(k0, n, t_len, off, tab_ref, h, num_buckets, max_distance):
    k = k0 + jax.lax.broadcasted_iota(jnp.int32, (1, n), 1)
    d = jnp.maximum(t_len - 1 + off - k, 0)
    max_exact = num_buckets // 2
    is_small = d < max_exact
    d_f = d.astype(jnp.float32)
    large = max_exact + (
        jnp.log(d_f / max_exact)
        / math.log(max_distance / max_exact)
        * (num_buckets - max_exact)
    ).astype(jnp.int32)
    large = jnp.minimum(large, num_buckets - 1)
    bucket = jnp.where(is_small, d, large)
    val = jnp.zeros((1, n), jnp.float32)
    for b in range(num_buckets):
        val = jnp.where(bucket == b, tab_ref[b, h], val)
    return val


def _bias_tile_kernel(off_ref, tab_ref, out_ref, diag_ref, *, t_len, num_buckets,
                      max_distance):
    ti = out_ref.shape[2]
    s_len = out_ref.shape[3]
    strip_len = ti + s_len - 1
    strip_pad = ((strip_len + 127) // 128) * 128
    h = pl.program_id(0)
    t = pl.program_id(1)
    off = off_ref[0]
    w0 = (t_len - ti) - t * ti
    vals = functools.partial(
        _diag_vals,
        t_len=t_len,
        off=off,
        tab_ref=tab_ref,
        h=h,
        num_buckets=num_buckets,
        max_distance=max_distance,
    )

    @pl.when(t == 0)
    def _():
        diag_ref[0:1, t_len - ti : t_len - ti + strip_pad] = vals(
            t_len - ti, strip_pad
        )

    @pl.when(t != 0)
    def _():
        diag_ref[0:1, pl.ds(w0, ti)] = vals(w0, ti)

    strip = diag_ref[0:1, pl.ds(w0, strip_pad)]

    rows = []
    for ii in range(ti):
        st = ti - 1 - ii
        rows.append(jax.lax.slice(strip, (0, st), (1, st + s_len)))
        if len(rows) == 8:
            out_ref[0, 0, ii - 7 : ii + 1, :] = jnp.concatenate(rows, axis=0)
            rows = []


def kernel(query, key, offset, rel_bias_table):
    t_len = query.shape[1]
    s_len = key.shape[1]
    num_buckets, num_heads = rel_bias_table.shape
    off = jnp.asarray(offset, jnp.int32).reshape(1)
    strip_pad = ((_TI + s_len - 1 + 127) // 128) * 128

    body = functools.partial(
        _bias_tile_kernel, t_len=t_len, num_buckets=num_buckets, max_distance=128
    )
    return pl.pallas_call(
        body,
        grid=(num_heads, t_len // _TI),
        in_specs=[
            pl.BlockSpec(memory_space=pltpu.SMEM),
            pl.BlockSpec(memory_space=pltpu.SMEM),
        ],
        out_specs=pl.BlockSpec((1, 1, _TI, s_len), lambda h, t: (0, h, t, 0)),
        out_shape=jax.ShapeDtypeStruct((1, num_heads, t_len, s_len), jnp.float32),
        scratch_shapes=[pltpu.VMEM((1, (t_len - _TI) + strip_pad), jnp.float32)],
    )(off, rel_bias_table)


# speedup vs baseline: 83.1195x; 1.0706x over previous
import functools
import math

import jax
import jax.numpy as jnp
from jax.experimental import pallas as pl
from jax.experimental.pallas import tpu as pltpu

_TI = 128


def _diag_vals(m0, n, t_len, off, tab_ref, h, num_buckets, max_distance):
    k = (m0 - 128) + jax.lax.broadcasted_iota(jnp.int32, (1, n), 1)
    d = jnp.maximum(t_len - 1 + off - k, 0)
    max_exact = num_buckets // 2
    is_small = d < max_exact
    d_f = d.astype(jnp.float32)
    large = max_exact + (
        jnp.log(d_f / max_exact)
        / math.log(max_distance / max_exact)
        * (num_buckets - max_exact)
    ).astype(jnp.int32)
    large = jnp.minimum(large, num_buckets - 1)
    bucket = jnp.where(is_small, d, large)
    val = jnp.zeros((1, n), jnp.float32)
    for b in range(num_buckets):
        val = jnp.where(bucket == b, tab_ref[b, h], val)
    return val


def _bias_tile_kernel(off_ref, tab_ref, out_ref, diag_ref, diag8_ref, *, t_len,
                      num_buckets, max_distance):
    ti = out_ref.shape[2]
    s_len = out_ref.shape[3]
    h = pl.program_id(0)
    t = pl.program_id(1)
    off = off_ref[0]
    w0 = (t_len - ti) - t * ti
    vals = functools.partial(
        _diag_vals,
        t_len=t_len,
        off=off,
        tab_ref=tab_ref,
        h=h,
        num_buckets=num_buckets,
        max_distance=max_distance,
    )

    @pl.when(t == 0)
    def _():
        m_lo = t_len
        n = ((2 * ti + s_len + 127) // 128) * 128
        diag_ref[0:1, m_lo : m_lo + n + 128] = vals(m_lo, n + 128)
        big = diag_ref[0:1, m_lo : m_lo + n + 128]
        for r in range(8):
            diag8_ref[r : r + 1, m_lo : m_lo + n] = jax.lax.slice(
                big, (0, 7 - r), (1, 7 - r + n)
            )

    @pl.when(t != 0)
    def _():
        m0 = w0 + 128
        diag_ref[0:1, pl.ds(m0, ti)] = vals(m0, ti)
        win = diag_ref[0:1, pl.ds(m0, 2 * ti)]
        for r in range(8):
            diag8_ref[r : r + 1, pl.ds(m0, ti)] = jax.lax.slice(
                win, (0, 7 - r), (1, 7 - r + ti)
            )

    win_len = ((120 + s_len + 127) // 128) * 128
    tile8 = diag8_ref[:, pl.ds(w0 + ti, win_len)]
    for g in range(ti // 8):
        x = 120 - 8 * g
        out_ref[0, 0, 8 * g : 8 * g + 8, :] = jax.lax.slice(
            tile8, (0, x), (8, x + s_len)
        )


def kernel(query, key, offset, rel_bias_table):
    t_len = query.shape[1]
    s_len = key.shape[1]
    num_buckets, num_heads = rel_bias_table.shape
    off = jnp.asarray(offset, jnp.int32).reshape(1)
    diag_len = ((128 + t_len + 2 * _TI + s_len + 255) // 128) * 128

    body = functools.partial(
        _bias_tile_kernel, t_len=t_len, num_buckets=num_buckets, max_distance=128
    )
    return pl.pallas_call(
        body,
        grid=(num_heads, t_len // _TI),
        in_specs=[
            pl.BlockSpec(memory_space=pltpu.SMEM),
            pl.BlockSpec(memory_space=pltpu.SMEM),
        ],
        out_specs=pl.BlockSpec((1, 1, _TI, s_len), lambda h, t: (0, h, t, 0)),
        out_shape=jax.ShapeDtypeStruct((1, num_heads, t_len, s_len), jnp.float32),
        scratch_shapes=[
            pltpu.VMEM((1, diag_len), jnp.float32),
            pltpu.VMEM((8, diag_len), jnp.float32),
        ],
    )(off, rel_bias_table)
